# Initial kernel scaffold; baseline (speedup 1.0000x reference)
import jax
import jax.numpy as jnp
from jax.experimental import pallas as pl


def kernel(classifications, regressions, anchors, annotations):
    raise NotImplementedError("write your pallas kernel here")


# speedup vs baseline: 10.8649x; 10.8649x over previous
import functools

import jax
import jax.numpy as jnp
from jax import lax
from jax.experimental import pallas as pl
from jax.experimental.pallas import tpu as pltpu


def _body(NB, Bn, A, cls_ref, reg_ref, anc_ref, ann_ref, out_ref, acc_ref):
    b = pl.program_id(0)
    i = pl.program_id(1)
    M = ann_ref.shape[1]
    C = cls_ref.shape[2]
    BA = cls_ref.shape[1]

    cls = cls_ref[0]
    p = jnp.where(cls > 1e-4, cls, 1e-4)
    p = jnp.where(p < 1.0 - 1e-4, p, 1.0 - 1e-4)
    ann = ann_ref[0]
    bx1 = ann[:, 0:1]
    by1 = ann[:, 1:2]
    bx2 = ann[:, 2:3]
    by2 = ann[:, 3:4]
    lab = ann[:, 4:5]
    ax1 = anc_ref[0:1, :]
    ay1 = anc_ref[1:2, :]
    ax2 = anc_ref[2:3, :]
    ay2 = anc_ref[3:4, :]

    iw = jnp.maximum(jnp.minimum(ax2, bx2) - jnp.maximum(ax1, bx1), 0.0)
    ih = jnp.maximum(jnp.minimum(ay2, by2) - jnp.maximum(ay1, by1), 0.0)
    inter = iw * ih
    area_b = (bx2 - bx1) * (by2 - by1)
    area_a = (ax2 - ax1) * (ay2 - ay1)
    ua = jnp.maximum(area_a + area_b - inter, 1e-8)
    iou = inter / ua

    iou_max = jnp.max(iou, axis=0, keepdims=True)
    m_iota = lax.broadcasted_iota(jnp.int32, (M, BA), 0)
    iou_arg = jnp.min(jnp.where(iou == iou_max, m_iota, M), axis=0,
                      keepdims=True)
    onehot = (m_iota == iou_arg).astype(jnp.float32)

    lane_a = lax.broadcasted_iota(jnp.int32, (1, BA), 1)
    alive = (i * BA + lane_a) < A
    pos = jnp.logical_and(iou_max >= 0.5, alive)
    neg = iou_max < 0.4
    posf = pos.astype(jnp.float32)
    valid = jnp.logical_and(jnp.logical_or(pos, neg), alive).astype(jnp.float32)
    npos_blk = jnp.sum(posf)

    f0 = (0.75 * p * p) * (-jnp.log(1.0 - p))
    s0 = lax.dot_general(valid, f0, (((1,), (0,)), ((), ())),
                         preferred_element_type=jnp.float32)
    cls_blk = jnp.sum(s0)

    c_iota = lax.broadcasted_iota(jnp.int32, (M, C), 1)
    sel = (c_iota == lab.astype(jnp.int32)).astype(jnp.float32)
    pcolsT = lax.dot_general(sel, p, (((1,), (1,)), ((), ())),
                             preferred_element_type=jnp.float32)
    q = jnp.sum(pcolsT * onehot, axis=0, keepdims=True)
    f0q = (0.75 * q * q) * (-jnp.log(1.0 - q))
    f1q = (0.25 * (1.0 - q) * (1.0 - q)) * (-jnp.log(q))
    cls_blk += jnp.sum(posf * (f1q - f0q))

    gx1 = jnp.sum(onehot * bx1, axis=0, keepdims=True)
    gy1 = jnp.sum(onehot * by1, axis=0, keepdims=True)
    gx2 = jnp.sum(onehot * bx2, axis=0, keepdims=True)
    gy2 = jnp.sum(onehot * by2, axis=0, keepdims=True)
    aw = ax2 - ax1
    ah = ay2 - ay1
    acx = ax1 + 0.5 * aw
    acy = ay1 + 0.5 * ah
    gwr = gx2 - gx1
    ghr = gy2 - gy1
    gcx = gx1 + 0.5 * gwr
    gcy = gy1 + 0.5 * ghr
    gw = jnp.maximum(gwr, 1.0)
    gh = jnp.maximum(ghr, 1.0)
    tdx = ((gcx - acx) / aw) / 0.1
    tdy = ((gcy - acy) / ah) / 0.1
    tdw = jnp.log(gw / aw) / 0.2
    tdh = jnp.log(gh / ah) / 0.2
    r = reg_ref[0]

    def _sl1(d):
        return jnp.where(d <= 1.0 / 9.0, 4.5 * d * d, d - 1.0 / 18.0)

    rsum = (_sl1(jnp.abs(tdx - r[0:1, :])) + _sl1(jnp.abs(tdy - r[1:2, :]))
            + _sl1(jnp.abs(tdw - r[2:3, :])) + _sl1(jnp.abs(tdh - r[3:4, :])))
    reg_blk = jnp.sum(rsum * posf)

    lane = lax.broadcasted_iota(jnp.int32, (1, 128), 1)

    @pl.when(jnp.logical_and(b == 0, i == 0))
    def _init_out():
        out_ref[...] = jnp.zeros_like(out_ref)

    @pl.when(i == 0)
    def _init_acc():
        acc_ref[...] = jnp.zeros_like(acc_ref)

    acc_ref[...] += (jnp.where(lane == 0, cls_blk, 0.0)
                     + jnp.where(lane == 1, reg_blk, 0.0)
                     + jnp.where(lane == 2, npos_blk, 0.0))

    @pl.when(i == NB - 1)
    def _finalize():
        acc = acc_ref[...]
        csum = jnp.sum(jnp.where(lane == 0, acc, 0.0))
        rsum_t = jnp.sum(jnp.where(lane == 1, acc, 0.0))
        npv = jnp.sum(jnp.where(lane == 2, acc, 0.0))
        npc = jnp.maximum(npv, 1.0)
        cl = csum / npc
        rl = jnp.where(npv > 0.0, rsum_t / (npc * 4.0), 0.0)
        out_ref[...] += (jnp.where(lane == 0, cl / Bn, 0.0)
                         + jnp.where(lane == 1, rl / Bn, 0.0))


def kernel(classifications, regressions, anchors, annotations):
    Bn, A, C = classifications.shape
    M = annotations.shape[1]
    BA = 2048 if A >= 2048 else ((A + 7) // 8) * 8
    NB = -(-A // BA)
    A_pad = NB * BA
    anchors_t = anchors[0].T
    reg_t = jnp.swapaxes(regressions, 1, 2)
    if A_pad != A:
        pad_box = jnp.tile(jnp.array([[0.0], [0.0], [1.0], [1.0]],
                                     dtype=anchors_t.dtype), (1, A_pad - A))
        anchors_t = jnp.concatenate([anchors_t, pad_box], axis=1)
        reg_t = jnp.pad(reg_t, ((0, 0), (0, 0), (0, A_pad - A)))

    out = pl.pallas_call(
        functools.partial(_body, NB, Bn, A),
        grid=(Bn, NB),
        in_specs=[
            pl.BlockSpec((1, BA, C), lambda b, i: (b, i, 0)),
            pl.BlockSpec((1, 4, BA), lambda b, i: (b, 0, i)),
            pl.BlockSpec((4, BA), lambda b, i: (0, i)),
            pl.BlockSpec((1, M, 5), lambda b, i: (b, 0, 0)),
        ],
        out_specs=pl.BlockSpec((1, 128), lambda b, i: (0, 0)),
        out_shape=jax.ShapeDtypeStruct((1, 128), jnp.float32),
        scratch_shapes=[pltpu.VMEM((1, 128), jnp.float32)],
    )(classifications, reg_t, anchors_t, annotations)
    return (out[0, 0:1], out[0, 1:2])
